# baseline (speedup 1.0000x reference)
import jax
import jax.numpy as jnp
from jax import lax
from jax.experimental import pallas as pl
from jax.experimental.pallas import tpu as pltpu
from jax.experimental.pallas import tpu_sc as plsc

MASK_PROB = 0.15
MASK_TOKEN = 103
LABEL_IGNORE = -100
RAND_THRESH_BITS = 0x3E19999A

ROWS = 4096
COLS = 2048
NC, NS, LANES = 2, 16, 16
NW = NC * NS
ROWS_W = ROWS // NW
CR = 8
NCH = ROWS_W // CR
VECS = CR * COLS // LANES

TC_BLOCK_ROWS = 256


def _sc_labels_body(ids_hbm, rand_hbm, l_hbm,
                    buf_ids0, buf_rnd0, buf_ids1, buf_rnd1,
                    si0, sr0, si1, sr1, sol0, sol1):
    wid = lax.axis_index("s") * NC + lax.axis_index("c")
    base = wid * ROWS_W

    bufs = ((buf_ids0, buf_rnd0, si0, sr0, sol0),
            (buf_ids1, buf_rnd1, si1, sr1, sol1))

    def start_in(c):
        ids_b, rnd_b, si, sr, _ = bufs[c % 2]
        r0 = base + c * CR
        h_i = pltpu.async_copy(ids_hbm.at[pl.ds(r0, CR)], ids_b, si)
        h_r = pltpu.async_copy(rand_hbm.at[pl.ds(r0, CR)], rnd_b, sr)
        return h_i, h_r

    out_handles = [None, None]
    in_handles = [None, None]

    in_handles[0] = start_in(0)
    for c in range(NCH):
        b = c % 2
        ids_b, rnd_b, _, _, sol = bufs[b]
        if c + 1 < NCH:
            nb = (c + 1) % 2
            if out_handles[nb] is not None:
                out_handles[nb].wait()
                out_handles[nb] = None
            in_handles[nb] = start_in(c + 1)
        in_handles[b][0].wait()
        in_handles[b][1].wait()

        @plsc.parallel_loop(0, VECS, unroll=8)
        def _(i):
            r = i >> 7
            sl = pl.ds((i & 127) * LANES, LANES)
            ids = ids_b[r, sl]
            rnd = rnd_b[r, sl]
            cond = (rnd < RAND_THRESH_BITS) | (ids == MASK_TOKEN)
            ids_b[r, sl] = jnp.where(cond, ids, jnp.int32(LABEL_IGNORE))

        r0 = base + c * CR
        out_handles[b] = pltpu.async_copy(ids_b, l_hbm.at[pl.ds(r0, CR)], sol)

    for b in range(2):
        if out_handles[b] is not None:
            out_handles[b].wait()


def _sc_labels(ids, rand_bits):
    mesh = plsc.VectorSubcoreMesh(core_axis_name="c", subcore_axis_name="s",
                                  num_cores=NC, num_subcores=NS)
    run = pl.kernel(
        _sc_labels_body,
        out_type=jax.ShapeDtypeStruct((ROWS, COLS), jnp.int32),
        mesh=mesh,
        scratch_types=[pltpu.VMEM((CR, COLS), jnp.int32) for _ in range(4)]
        + [pltpu.SemaphoreType.DMA for _ in range(6)],
    )
    return run(ids, rand_bits)


def _tc_masked_kernel(ids_ref, rand_ref, m_ref):
    m_ref[...] = jnp.where(rand_ref[...] < MASK_PROB,
                           jnp.int32(MASK_TOKEN), ids_ref[...])


def _tc_masked(ids, rand_vals):
    grid = (ROWS // TC_BLOCK_ROWS,)
    spec = pl.BlockSpec((TC_BLOCK_ROWS, COLS), lambda i: (i, 0))
    return pl.pallas_call(
        _tc_masked_kernel,
        grid=grid,
        in_specs=[spec, spec],
        out_specs=spec,
        out_shape=jax.ShapeDtypeStruct((ROWS, COLS), jnp.int32),
    )(ids, rand_vals)


def kernel(input_ids, rand_vals):
    rand_bits = lax.bitcast_convert_type(rand_vals, jnp.int32)
    labels = _sc_labels(input_ids, rand_bits)
    masked = _tc_masked(input_ids, rand_vals)
    return masked, labels


# speedup vs baseline: 2.4818x; 2.4818x over previous
import jax
import jax.numpy as jnp
from jax.experimental import pallas as pl

MASK_PROB = 0.15
MASK_TOKEN = 103

BLOCK_ROWS = 512


def _mask_kernel(ids_ref, rand_ref, masked_ref, labels_ref):
    ids = ids_ref[...]
    rand = rand_ref[...]
    mask = rand < MASK_PROB
    masked = jnp.where(mask, jnp.int32(MASK_TOKEN), ids)
    masked_ref[...] = masked
    labels_ref[...] = jnp.where(masked == MASK_TOKEN, ids, jnp.int32(-100))


def kernel(input_ids, rand_vals):
    n_rows, n_cols = input_ids.shape
    grid = (n_rows // BLOCK_ROWS,)
    spec = pl.BlockSpec((BLOCK_ROWS, n_cols), lambda i: (i, 0))
    out_shape = jax.ShapeDtypeStruct(input_ids.shape, input_ids.dtype)
    masked, labels = pl.pallas_call(
        _mask_kernel,
        grid=grid,
        in_specs=[spec, spec],
        out_specs=[spec, spec],
        out_shape=[out_shape, out_shape],
    )(input_ids, rand_vals)
    return masked, labels


# speedup vs baseline: 2.4821x; 1.0001x over previous
import jax
import jax.numpy as jnp
from jax.experimental import pallas as pl

MASK_PROB = 0.15
MASK_TOKEN = 103

BLOCK_ROWS = 1024
BLOCK_COLS = 1024


def _mask_kernel(ids_ref, rand_ref, masked_ref, labels_ref):
    ids = ids_ref[...]
    rand = rand_ref[...]
    mask = rand < MASK_PROB
    masked = jnp.where(mask, jnp.int32(MASK_TOKEN), ids)
    masked_ref[...] = masked
    labels_ref[...] = jnp.where(masked == MASK_TOKEN, ids, jnp.int32(-100))


def kernel(input_ids, rand_vals):
    n_rows, n_cols = input_ids.shape
    grid = (n_rows // BLOCK_ROWS, n_cols // BLOCK_COLS)
    spec = pl.BlockSpec((BLOCK_ROWS, BLOCK_COLS), lambda i, j: (i, j))
    out_shape = jax.ShapeDtypeStruct(input_ids.shape, input_ids.dtype)
    masked, labels = pl.pallas_call(
        _mask_kernel,
        grid=grid,
        in_specs=[spec, spec],
        out_specs=[spec, spec],
        out_shape=[out_shape, out_shape],
    )(input_ids, rand_vals)
    return masked, labels


# speedup vs baseline: 2.4841x; 1.0008x over previous
import jax
import jax.numpy as jnp
from jax.experimental import pallas as pl

MASK_PROB = 0.15
MASK_TOKEN = 103

BLOCK_ROWS = 512


def _mask_kernel(ids_ref, rand_ref, masked_ref, labels_ref):
    ids = ids_ref[...]
    rand = rand_ref[...]
    mask = rand < MASK_PROB
    masked = jnp.where(mask, jnp.int32(MASK_TOKEN), ids)
    masked_ref[...] = masked
    labels_ref[...] = jnp.where(masked == MASK_TOKEN, ids, jnp.int32(-100))


def kernel(input_ids, rand_vals):
    n_rows, n_cols = input_ids.shape
    grid = (n_rows // BLOCK_ROWS,)
    in_spec = pl.BlockSpec((BLOCK_ROWS, n_cols), lambda i: (i, 0))
    out_spec = pl.BlockSpec((BLOCK_ROWS, n_cols), lambda i: (i, 0))
    out_shape = jax.ShapeDtypeStruct(input_ids.shape, input_ids.dtype)
    masked, labels = pl.pallas_call(
        _mask_kernel,
        grid=grid,
        in_specs=[in_spec, in_spec],
        out_specs=[out_spec, out_spec],
        out_shape=[out_shape, out_shape],
    )(input_ids, rand_vals)
    return masked, labels
